# baseline (speedup 1.0000x reference)
import jax
import jax.numpy as jnp
from jax import lax
from jax.experimental import pallas as pl
from jax.experimental.pallas import tpu as pltpu
from jax.experimental.pallas import tpu_sc as plsc

N_RELATION = 8
EMB = 1024
NTOK = 4096
BM = 256
NBE = NTOK // BM + N_RELATION - 1
BEPAD = 48
NPAD = NBE * BM
NC, NS, L = 2, 16, 16
NW = NC * NS
CHUNK = NTOK // NW
ROWBUF = 64


def _it16():
    return lax.broadcasted_iota(jnp.int32, (L,), 0)


def _route_body(rel_hbm, x_hbm, xs_hbm, pos_hbm, be_hbm, cnt_hbm,
                relv, cbuf, table, posbuf, xbuf, bexbuf, sem):
    c = lax.axis_index("c")
    s = lax.axis_index("s")
    wid = 2 * s + c
    it = _it16()

    for q in range(2):
        w = 2 * s + q
        pltpu.sync_copy(rel_hbm.at[pl.ds(w * CHUNK, CHUNK)], relv)
        counts = jnp.zeros((L,), jnp.int32)
        for j in range(CHUNK // L):
            v = relv[pl.ds(j * L, L)]
            for r in range(N_RELATION):
                cnt = jnp.sum(jnp.where(v == r, 1, 0))
                counts = counts + jnp.where(it == r, cnt, 0)
        cbuf[...] = counts
        pltpu.sync_copy(cbuf, cnt_hbm.at[w])
    plsc.subcore_barrier()

    pltpu.sync_copy(cnt_hbm, table)
    totals = jnp.zeros((L,), jnp.int32)
    mine = jnp.zeros((L,), jnp.int32)
    for w in range(NW):
        row = table[w]
        totals = totals + row
        mine = mine + jnp.where(jnp.full((L,), w, jnp.int32) < wid, row, 0)
    nblk = (totals + (BM - 1)) // BM
    incl = plsc.cumsum(nblk)
    bstart = incl - nblk
    base = bstart * BM + mine

    @pl.when(jnp.logical_and(c == 0, s == 0))
    def _():
        for k in range(BEPAD // L):
            bvec = it + (k * L)
            ev = jnp.full((L,), -1, jnp.int32)
            for r in range(N_RELATION):
                bs_r = jnp.sum(jnp.where(it == r, bstart, 0))
                ev = ev + jnp.where(bs_r <= bvec, 1, 0)
            bexbuf[pl.ds(k * L, L)] = ev
        pltpu.sync_copy(bexbuf, be_hbm)

    pltpu.sync_copy(rel_hbm.at[pl.ds(wid * CHUNK, CHUNK)], relv)
    for j in range(CHUNK // L):
        v = relv[pl.ds(j * L, L)]
        posv = jnp.zeros((L,), jnp.int32)
        for r in range(N_RELATION):
            m = v == r
            ones = jnp.where(m, 1, 0)
            rk = plsc.cumsum(ones) - 1
            bs = jnp.sum(jnp.where(it == r, base, 0))
            posv = posv + jnp.where(m, bs + rk, 0)
            cnt = jnp.sum(ones)
            base = base + jnp.where(it == r, cnt, 0)
        posbuf[j // 4, pl.ds((j % 4) * L, L)] = posv
    pltpu.sync_copy(posbuf, pos_hbm.at[pl.ds(wid * 2, 2)])

    for q in range(CHUNK // ROWBUF):
        pltpu.sync_copy(x_hbm.at[pl.ds(wid * CHUNK + q * ROWBUF, ROWBUF)], xbuf)
        pltpu.async_copy(xbuf, xs_hbm.at[posbuf.at[q]], sem).wait()


def _combine_body(ys_hbm, pos_hbm, out_hbm, posbuf, ybuf, sem):
    c = lax.axis_index("c")
    s = lax.axis_index("s")
    wid = 2 * s + c
    pltpu.sync_copy(pos_hbm.at[pl.ds(wid * 2, 2)], posbuf)
    for q in range(CHUNK // ROWBUF):
        pltpu.async_copy(ys_hbm.at[posbuf.at[q]], ybuf, sem).wait()
        pltpu.sync_copy(ybuf, out_hbm.at[pl.ds(wid * CHUNK + q * ROWBUF, ROWBUF)])


def _mm_body(be_ref, x_ref, w_ref, y_ref, wbf_ref):
    i = pl.program_id(0)
    changed = jnp.logical_or(i == 0, be_ref[i] != be_ref[jnp.maximum(i - 1, 0)])

    @pl.when(changed)
    def _():
        wbf_ref[...] = w_ref[0].astype(jnp.bfloat16)

    y_ref[...] = lax.dot_general(x_ref[...].astype(jnp.bfloat16), wbf_ref[...],
                                 (((1,), (1,)), ((), ())),
                                 preferred_element_type=jnp.float32)


_STAGE = 3


def kernel(node_emb, relation, W):
    mesh = plsc.VectorSubcoreMesh(core_axis_name="c", subcore_axis_name="s")
    route = pl.kernel(
        _route_body,
        out_type=[
            jax.ShapeDtypeStruct((NPAD, EMB), jnp.float32),
            jax.ShapeDtypeStruct((NTOK // ROWBUF, ROWBUF), jnp.int32),
            jax.ShapeDtypeStruct((BEPAD,), jnp.int32),
            jax.ShapeDtypeStruct((NW, L), jnp.int32),
        ],
        mesh=mesh,
        scratch_types=[
            pltpu.VMEM((CHUNK,), jnp.int32),
            pltpu.VMEM((L,), jnp.int32),
            pltpu.VMEM((NW, L), jnp.int32),
            pltpu.VMEM((CHUNK // ROWBUF, ROWBUF), jnp.int32),
            pltpu.VMEM((ROWBUF, EMB), jnp.float32),
            pltpu.VMEM((BEPAD,), jnp.int32),
            pltpu.SemaphoreType.DMA,
        ],
        compiler_params=pltpu.CompilerParams(needs_layout_passes=False),
    )
    xs, pos, be, _ = route(relation, node_emb)
    if _STAGE == 1:
        return xs

    ys = pl.pallas_call(
        _mm_body,
        grid_spec=pltpu.PrefetchScalarGridSpec(
            num_scalar_prefetch=1,
            grid=(NBE,),
            in_specs=[
                pl.BlockSpec((BM, EMB), lambda i, be_ref: (i, 0)),
                pl.BlockSpec((1, EMB, EMB), lambda i, be_ref: (be_ref[i], 0, 0)),
            ],
            out_specs=pl.BlockSpec((BM, EMB), lambda i, be_ref: (i, 0)),
            scratch_shapes=[pltpu.VMEM((EMB, EMB), jnp.bfloat16)],
        ),
        out_shape=jax.ShapeDtypeStruct((NPAD, EMB), jnp.float32),
    )(be, xs, W)
    if _STAGE == 2:
        return ys

    combine = pl.kernel(
        _combine_body,
        out_type=jax.ShapeDtypeStruct((NTOK, EMB), jnp.float32),
        mesh=mesh,
        scratch_types=[
            pltpu.VMEM((CHUNK // ROWBUF, ROWBUF), jnp.int32),
            pltpu.VMEM((ROWBUF, EMB), jnp.float32),
            pltpu.SemaphoreType.DMA,
        ],
        compiler_params=pltpu.CompilerParams(needs_layout_passes=False),
    )
    return combine(ys, pos)


# speedup vs baseline: 1.0334x; 1.0334x over previous
import jax
import jax.numpy as jnp
from jax import lax
from jax.experimental import pallas as pl
from jax.experimental.pallas import tpu as pltpu
from jax.experimental.pallas import tpu_sc as plsc

N_RELATION = 8
EMB = 1024
NTOK = 4096
BM = 512
NBE = NTOK // BM + N_RELATION - 1
BEPAD = 48
NPAD = NBE * BM
NC, NS, L = 2, 16, 16
NW = NC * NS
CHUNK = NTOK // NW
ROWBUF = 64


def _it16():
    return lax.broadcasted_iota(jnp.int32, (L,), 0)


def _route_body(rel_hbm, x_hbm, xs_hbm, pos_hbm, be_hbm, cnt_hbm,
                relv, cbuf, table, posbuf, xbuf, bexbuf, sem):
    c = lax.axis_index("c")
    s = lax.axis_index("s")
    wid = 2 * s + c
    it = _it16()

    for q in range(2):
        w = 2 * s + q
        pltpu.sync_copy(rel_hbm.at[pl.ds(w * CHUNK, CHUNK)], relv)
        counts = jnp.zeros((L,), jnp.int32)
        for j in range(CHUNK // L):
            v = relv[pl.ds(j * L, L)]
            for r in range(N_RELATION):
                cnt = jnp.sum(jnp.where(v == r, 1, 0))
                counts = counts + jnp.where(it == r, cnt, 0)
        cbuf[...] = counts
        pltpu.sync_copy(cbuf, cnt_hbm.at[w])
    plsc.subcore_barrier()

    pltpu.sync_copy(cnt_hbm, table)
    totals = jnp.zeros((L,), jnp.int32)
    mine = jnp.zeros((L,), jnp.int32)
    for w in range(NW):
        row = table[w]
        totals = totals + row
        mine = mine + jnp.where(jnp.full((L,), w, jnp.int32) < wid, row, 0)
    nblk = (totals + (BM - 1)) // BM
    incl = plsc.cumsum(nblk)
    bstart = incl - nblk
    base = bstart * BM + mine

    @pl.when(jnp.logical_and(c == 0, s == 0))
    def _():
        for k in range(BEPAD // L):
            bvec = it + (k * L)
            ev = jnp.full((L,), -1, jnp.int32)
            for r in range(N_RELATION):
                bs_r = jnp.sum(jnp.where(it == r, bstart, 0))
                ev = ev + jnp.where(bs_r <= bvec, 1, 0)
            bexbuf[pl.ds(k * L, L)] = ev
        pltpu.sync_copy(bexbuf, be_hbm)

    pltpu.sync_copy(rel_hbm.at[pl.ds(wid * CHUNK, CHUNK)], relv)
    for j in range(CHUNK // L):
        v = relv[pl.ds(j * L, L)]
        posv = jnp.zeros((L,), jnp.int32)
        for r in range(N_RELATION):
            m = v == r
            ones = jnp.where(m, 1, 0)
            rk = plsc.cumsum(ones) - 1
            bs = jnp.sum(jnp.where(it == r, base, 0))
            posv = posv + jnp.where(m, bs + rk, 0)
            cnt = jnp.sum(ones)
            base = base + jnp.where(it == r, cnt, 0)
        posbuf[j // 4, pl.ds((j % 4) * L, L)] = posv
    pltpu.sync_copy(posbuf, pos_hbm.at[pl.ds(wid * 2, 2)])

    for q in range(CHUNK // ROWBUF):
        pltpu.sync_copy(x_hbm.at[pl.ds(wid * CHUNK + q * ROWBUF, ROWBUF)], xbuf)
        pltpu.async_copy(xbuf, xs_hbm.at[posbuf.at[q]], sem).wait()


def _combine_body(ys_hbm, pos_hbm, out_hbm, posbuf, ybuf, sem):
    c = lax.axis_index("c")
    s = lax.axis_index("s")
    wid = 2 * s + c
    pltpu.sync_copy(pos_hbm.at[pl.ds(wid * 2, 2)], posbuf)
    for q in range(CHUNK // ROWBUF):
        pltpu.async_copy(ys_hbm.at[posbuf.at[q]], ybuf, sem).wait()
        pltpu.sync_copy(ybuf, out_hbm.at[pl.ds(wid * CHUNK + q * ROWBUF, ROWBUF)])


def _mm_body(be_ref, x_ref, w_ref, y_ref, wbf_ref):
    i = pl.program_id(0)
    changed = jnp.logical_or(i == 0, be_ref[i] != be_ref[jnp.maximum(i - 1, 0)])

    @pl.when(changed)
    def _():
        wbf_ref[...] = w_ref[0].astype(jnp.bfloat16)

    y_ref[...] = lax.dot_general(x_ref[...].astype(jnp.bfloat16), wbf_ref[...],
                                 (((1,), (1,)), ((), ())),
                                 preferred_element_type=jnp.float32)


_STAGE = 3


def kernel(node_emb, relation, W):
    mesh = plsc.VectorSubcoreMesh(core_axis_name="c", subcore_axis_name="s")
    route = pl.kernel(
        _route_body,
        out_type=[
            jax.ShapeDtypeStruct((NPAD, EMB), jnp.float32),
            jax.ShapeDtypeStruct((NTOK // ROWBUF, ROWBUF), jnp.int32),
            jax.ShapeDtypeStruct((BEPAD,), jnp.int32),
            jax.ShapeDtypeStruct((NW, L), jnp.int32),
        ],
        mesh=mesh,
        scratch_types=[
            pltpu.VMEM((CHUNK,), jnp.int32),
            pltpu.VMEM((L,), jnp.int32),
            pltpu.VMEM((NW, L), jnp.int32),
            pltpu.VMEM((CHUNK // ROWBUF, ROWBUF), jnp.int32),
            pltpu.VMEM((ROWBUF, EMB), jnp.float32),
            pltpu.VMEM((BEPAD,), jnp.int32),
            pltpu.SemaphoreType.DMA,
        ],
        compiler_params=pltpu.CompilerParams(needs_layout_passes=False),
    )
    xs, pos, be, _ = route(relation, node_emb)
    if _STAGE == 1:
        return xs

    ys = pl.pallas_call(
        _mm_body,
        grid_spec=pltpu.PrefetchScalarGridSpec(
            num_scalar_prefetch=1,
            grid=(NBE,),
            in_specs=[
                pl.BlockSpec((BM, EMB), lambda i, be_ref: (i, 0)),
                pl.BlockSpec((1, EMB, EMB), lambda i, be_ref: (be_ref[i], 0, 0)),
            ],
            out_specs=pl.BlockSpec((BM, EMB), lambda i, be_ref: (i, 0)),
            scratch_shapes=[pltpu.VMEM((EMB, EMB), jnp.bfloat16)],
        ),
        out_shape=jax.ShapeDtypeStruct((NPAD, EMB), jnp.float32),
    )(be, xs, W)
    if _STAGE == 2:
        return ys

    combine = pl.kernel(
        _combine_body,
        out_type=jax.ShapeDtypeStruct((NTOK, EMB), jnp.float32),
        mesh=mesh,
        scratch_types=[
            pltpu.VMEM((CHUNK // ROWBUF, ROWBUF), jnp.int32),
            pltpu.VMEM((ROWBUF, EMB), jnp.float32),
            pltpu.SemaphoreType.DMA,
        ],
        compiler_params=pltpu.CompilerParams(needs_layout_passes=False),
    )
    return combine(ys, pos)
